# baseline (speedup 1.0000x reference)
import functools

import jax
import jax.numpy as jnp
from jax import lax
from jax.experimental import pallas as pl
from jax.experimental.pallas import tpu as pltpu
from jax.experimental.pallas import tpu_sc as plsc


def _proj_body(tl_ref, ti_ref, wt_ref, wb_ref, b_ref, out_ref):
    acc = jnp.dot(tl_ref[...], wt_ref[...], preferred_element_type=jnp.float32)
    acc += jnp.dot(ti_ref[...], wb_ref[...], preferred_element_type=jnp.float32)
    out_ref[...] = (acc + b_ref[...]).astype(jnp.bfloat16)


def _project_tables(table_lang, table_img, W, b):
    V, d_lang = table_lang.shape
    d_img = table_img.shape[1]
    d_out = W.shape[1]
    w_top = W[:d_lang]
    w_bot = W[d_lang:]
    bv = 4096
    grid_n = pl.cdiv(V, bv)
    return pl.pallas_call(
        _proj_body,
        grid=(grid_n,),
        in_specs=[
            pl.BlockSpec((bv, d_lang), lambda i: (i, 0)),
            pl.BlockSpec((bv, d_img), lambda i: (i, 0)),
            pl.BlockSpec((d_lang, d_out), lambda i: (0, 0)),
            pl.BlockSpec((d_img, d_out), lambda i: (0, 0)),
            pl.BlockSpec((1, d_out), lambda i: (0, 0)),
        ],
        out_specs=pl.BlockSpec((bv, d_out), lambda i: (i, 0)),
        out_shape=jax.ShapeDtypeStruct((grid_n * bv, d_out), jnp.bfloat16),
    )(table_lang, table_img, w_top, w_bot, b.reshape(1, d_out))


@functools.lru_cache(maxsize=None)
def _make_gather(V, D, N):
    info = plsc.get_sparse_core_info()
    nw = info.num_cores * info.num_subcores
    per_w = N // nw
    nbuf = 4
    ch = 640
    while ch > 8 and (per_w % (ch * nbuf) or ch % 8):
        ch -= 8
    n_groups = per_w // ch // nbuf
    mesh = plsc.VectorSubcoreMesh(core_axis_name="c", subcore_axis_name="s")

    @functools.partial(
        pl.kernel,
        out_type=jax.ShapeDtypeStruct((N, D), jnp.bfloat16),
        mesh=mesh,
        scratch_types=[
            pltpu.VMEM((per_w,), jnp.int32),
            [pltpu.VMEM((ch, D), jnp.bfloat16) for _ in range(nbuf)],
            [pltpu.SemaphoreType.DMA for _ in range(nbuf)],
            [pltpu.SemaphoreType.DMA for _ in range(nbuf)],
        ],
        compiler_params=pltpu.CompilerParams(use_tc_tiling_on_sc=False),
    )
    def gather(idx_hbm, t_hbm, out_hbm, idx_v, bufs, gsems, ssems):
        wid = lax.axis_index("s") * info.num_cores + lax.axis_index("c")
        wbase = wid * per_w
        pltpu.sync_copy(idx_hbm.at[pl.ds(wbase, per_w)], idx_v)

        def gather_desc(c, b):
            src = t_hbm.at[idx_v.at[pl.ds(c * ch, ch)]]
            return pltpu.make_async_copy(src, bufs[b], gsems[b])

        def store_desc(c, b):
            dst = out_hbm.at[pl.ds(wbase + c * ch, ch)]
            return pltpu.make_async_copy(bufs[b], dst, ssems[b])

        def group(g, carry):
            c0 = g * nbuf
            for b in range(nbuf):

                @pl.when(g > 0)
                def _(b=b):
                    store_desc(c0 - nbuf + b, b).wait()

                gather_desc(c0 + b, b).start()
            for b in range(nbuf):
                gather_desc(c0 + b, b).wait()
                store_desc(c0 + b, b).start()
            return carry

        lax.fori_loop(0, n_groups, group, 0)
        for b in range(nbuf):
            store_desc((n_groups - 1) * nbuf + b, b).wait()

    return gather


def kernel(x, table_lang, table_img, W, b):
    B, L = x.shape
    d_out = W.shape[1]
    proj = _project_tables(table_lang, table_img, W, b)
    idx = x.reshape(B * L).astype(jnp.int32)
    gather = _make_gather(proj.shape[0], d_out, B * L)
    out = gather(idx, proj).reshape(B, L, d_out).astype(jnp.float32)
    mask = x > 0
    return (out, mask)


# speedup vs baseline: 1.4668x; 1.4668x over previous
import functools

import jax
import jax.numpy as jnp
from jax import lax
from jax.experimental import pallas as pl
from jax.experimental.pallas import tpu as pltpu
from jax.experimental.pallas import tpu_sc as plsc


def _proj_body(tl_ref, ti_ref, wt_ref, wb_ref, b_ref, out_ref):
    acc = jnp.dot(tl_ref[...], wt_ref[...], preferred_element_type=jnp.float32)
    acc += jnp.dot(ti_ref[...], wb_ref[...], preferred_element_type=jnp.float32)
    out_ref[...] = acc + b_ref[...]


def _project_tables(table_lang, table_img, W, b):
    V, d_lang = table_lang.shape
    d_img = table_img.shape[1]
    d_out = W.shape[1]
    w_top = W[:d_lang]
    w_bot = W[d_lang:]
    bv = 4096
    grid_n = pl.cdiv(V, bv)
    return pl.pallas_call(
        _proj_body,
        grid=(grid_n,),
        in_specs=[
            pl.BlockSpec((bv, d_lang), lambda i: (i, 0)),
            pl.BlockSpec((bv, d_img), lambda i: (i, 0)),
            pl.BlockSpec((d_lang, d_out), lambda i: (0, 0)),
            pl.BlockSpec((d_img, d_out), lambda i: (0, 0)),
            pl.BlockSpec((1, d_out), lambda i: (0, 0)),
        ],
        out_specs=pl.BlockSpec((bv, d_out), lambda i: (i, 0)),
        out_shape=jax.ShapeDtypeStruct((grid_n * bv, d_out), jnp.float32),
    )(table_lang, table_img, w_top, w_bot, b.reshape(1, d_out))


@functools.lru_cache(maxsize=None)
def _make_gather(V, D, N):
    info = plsc.get_sparse_core_info()
    nw = info.num_cores * info.num_subcores
    per_w = N // nw
    nbuf = 4
    ch = 320
    while ch > 8 and (per_w % (ch * nbuf) or ch % 8):
        ch -= 8
    n_groups = per_w // ch // nbuf
    mesh = plsc.VectorSubcoreMesh(core_axis_name="c", subcore_axis_name="s")

    @functools.partial(
        pl.kernel,
        out_type=jax.ShapeDtypeStruct((N, D), jnp.float32),
        mesh=mesh,
        scratch_types=[
            pltpu.VMEM((per_w,), jnp.int32),
            [pltpu.VMEM((ch, D), jnp.float32) for _ in range(nbuf)],
            [pltpu.SemaphoreType.DMA for _ in range(nbuf)],
            [pltpu.SemaphoreType.DMA for _ in range(nbuf)],
        ],
        compiler_params=pltpu.CompilerParams(use_tc_tiling_on_sc=False),
    )
    def gather(idx_hbm, t_hbm, out_hbm, idx_v, bufs, gsems, ssems):
        wid = lax.axis_index("s") * info.num_cores + lax.axis_index("c")
        wbase = wid * per_w
        pltpu.sync_copy(idx_hbm.at[pl.ds(wbase, per_w)], idx_v)

        def gather_desc(c, b):
            src = t_hbm.at[idx_v.at[pl.ds(c * ch, ch)]]
            return pltpu.make_async_copy(src, bufs[b], gsems[b])

        def store_desc(c, b):
            dst = out_hbm.at[pl.ds(wbase + c * ch, ch)]
            return pltpu.make_async_copy(bufs[b], dst, ssems[b])

        def group(g, carry):
            c0 = g * nbuf
            for b in range(nbuf):

                @pl.when(g > 0)
                def _(b=b):
                    store_desc(c0 - nbuf + b, b).wait()

                gather_desc(c0 + b, b).start()
            for b in range(nbuf):
                gather_desc(c0 + b, b).wait()
                store_desc(c0 + b, b).start()
            return carry

        lax.fori_loop(0, n_groups, group, 0)
        for b in range(nbuf):
            store_desc((n_groups - 1) * nbuf + b, b).wait()

    return gather


def kernel(x, table_lang, table_img, W, b):
    B, L = x.shape
    d_out = W.shape[1]
    proj = _project_tables(table_lang, table_img, W, b)
    idx = x.reshape(B * L).astype(jnp.int32)
    gather = _make_gather(proj.shape[0], d_out, B * L)
    out = gather(idx, proj).reshape(B, L, d_out)
    mask = x > 0
    return (out, mask)
